# Initial kernel scaffold; baseline (speedup 1.0000x reference)
import jax
import jax.numpy as jnp
from jax.experimental import pallas as pl


def kernel(x, edge_index, W1, b1, W2, b2):
    raise NotImplementedError("write your pallas kernel here")


# speedup vs baseline: 43.1525x; 43.1525x over previous
import functools

import jax
import jax.numpy as jnp
from jax import lax
from jax.experimental import pallas as pl
from jax.experimental.pallas import tpu as pltpu
from jax.experimental.pallas import tpu_sc as plsc

N_NODES = 10000
IN_DIM = 128
HID_DIM = 16
OUT_DIM = 16
N_EDGES = 320000

NCORES = 2
NSUB = 16
NW = NCORES * NSUB
CHUNK = 128
JCH = 79
E_PER_W = CHUNK * JCH
E_PAD = NW * E_PER_W
N_ACC = 10112
ROWS_PER_TILE = N_ACC // NSUB


def _sc_mesh():
    return plsc.VectorSubcoreMesh(core_axis_name="c", subcore_axis_name="s")


def _fill_rows(ref, nrows, value):
    row = jnp.full((16,), value, jnp.float32)

    def body(i, carry):
        ref[i, :] = row
        return carry

    lax.fori_loop(0, nrows, body, 0)


def _zero_acc_slice(zero_v, acc, tid):
    _fill_rows(zero_v, 8, 0.0)

    def body(i, carry):
        pltpu.sync_copy(zero_v, acc.at[pl.ds(tid * ROWS_PER_TILE + i * 8, 8)])
        return carry

    lax.fori_loop(0, ROWS_PER_TILE // 8, body, 0)


@functools.partial(
    pl.kernel,
    out_type=jax.ShapeDtypeStruct((NCORES, N_ACC, 16), jnp.float32),
    mesh=_sc_mesh(),
    scratch_types=[
        pltpu.VMEM((JCH, CHUNK), jnp.int32),
        pltpu.VMEM((CHUNK, 16), jnp.float32),
        pltpu.VMEM((8, 16), jnp.float32),
        pltpu.VMEM_SHARED((N_ACC, 16), jnp.float32),
    ],
)
def _sc_hist(dst_hbm, out_hbm, dst_v, ones_v, zero_v, acc):
    cid = lax.axis_index("c")
    tid = lax.axis_index("s")
    wid = cid * NSUB + tid
    pltpu.sync_copy(dst_hbm.at[wid], dst_v)
    _fill_rows(ones_v, CHUNK, 1.0)
    _zero_acc_slice(zero_v, acc, tid)
    plsc.subcore_barrier()

    def body(j, carry):
        pltpu.sync_copy(ones_v, acc.at[dst_v.at[j]], add=True)
        return carry

    lax.fori_loop(0, JCH, body, 0)
    plsc.subcore_barrier()
    pltpu.sync_copy(
        acc.at[pl.ds(tid * ROWS_PER_TILE, ROWS_PER_TILE)],
        out_hbm.at[cid, pl.ds(tid * ROWS_PER_TILE, ROWS_PER_TILE)],
    )


@functools.partial(
    pl.kernel,
    out_type=jax.ShapeDtypeStruct((NCORES, N_ACC, 16), jnp.float32),
    mesh=_sc_mesh(),
    scratch_types=[
        pltpu.VMEM((JCH, CHUNK), jnp.int32),
        pltpu.VMEM((JCH, CHUNK), jnp.int32),
        pltpu.VMEM((CHUNK, 16), jnp.float32),
        pltpu.VMEM((8, 16), jnp.float32),
        pltpu.VMEM_SHARED((N_ACC, 16), jnp.float32),
        pltpu.VMEM_SHARED((N_NODES, 16), jnp.float32),
        pltpu.SemaphoreType.DMA,
    ],
)
def _sc_agg(tbl_hbm, src_hbm, dst_hbm, out_hbm, src_v, dst_v, row_v, zero_v, acc,
            tbl_sp, sem):
    cid = lax.axis_index("c")
    tid = lax.axis_index("s")
    wid = cid * NSUB + tid
    pltpu.sync_copy(src_hbm.at[wid], src_v)
    pltpu.sync_copy(dst_hbm.at[wid], dst_v)
    @pl.when(tid < NSUB - 1)
    def _():
        pltpu.sync_copy(
            tbl_hbm.at[pl.ds(tid * ROWS_PER_TILE, ROWS_PER_TILE)],
            tbl_sp.at[pl.ds(tid * ROWS_PER_TILE, ROWS_PER_TILE)],
        )

    TAIL = N_NODES - (NSUB - 1) * ROWS_PER_TILE

    @pl.when(tid == NSUB - 1)
    def _():
        pltpu.sync_copy(
            tbl_hbm.at[pl.ds((NSUB - 1) * ROWS_PER_TILE, TAIL)],
            tbl_sp.at[pl.ds((NSUB - 1) * ROWS_PER_TILE, TAIL)],
        )

    _zero_acc_slice(zero_v, acc, tid)
    plsc.subcore_barrier()

    def body(j, carry):
        pltpu.async_copy(tbl_sp.at[src_v.at[j]], row_v, sem).wait()
        pltpu.sync_copy(row_v, acc.at[dst_v.at[j]], add=True)
        return carry

    lax.fori_loop(0, JCH, body, 0)
    plsc.subcore_barrier()
    pltpu.sync_copy(
        acc.at[pl.ds(tid * ROWS_PER_TILE, ROWS_PER_TILE)],
        out_hbm.at[cid, pl.ds(tid * ROWS_PER_TILE, ROWS_PER_TILE)],
    )


RB = 1000
GRID = N_NODES // RB


def _rowspec(width):
    return pl.BlockSpec((RB, width), lambda i: (i, 0))


def _mm1_body(x_ref, w_ref, h_ref):
    h_ref[...] = jnp.dot(x_ref[...], w_ref[...], preferred_element_type=jnp.float32)


def _tc_matmul1(x, W1):
    return pl.pallas_call(
        _mm1_body,
        grid=(GRID,),
        in_specs=[_rowspec(IN_DIM), pl.BlockSpec((IN_DIM, HID_DIM), lambda i: (0, 0))],
        out_specs=_rowspec(HID_DIM),
        out_shape=jax.ShapeDtypeStruct((N_NODES, HID_DIM), jnp.float32),
    )(x, W1)


def _scale1_body(p0_ref, p1_ref, h_ref, g_ref, dis_ref):
    deg = p0_ref[...] + p1_ref[...] + 1.0
    dis = lax.rsqrt(deg)
    dis_ref[...] = dis
    g_ref[...] = h_ref[...] * dis


def _tc_scale1(p0, p1, h1):
    return pl.pallas_call(
        _scale1_body,
        grid=(GRID,),
        in_specs=[_rowspec(16), _rowspec(16), _rowspec(HID_DIM)],
        out_specs=[_rowspec(HID_DIM), _rowspec(16)],
        out_shape=[
            jax.ShapeDtypeStruct((N_NODES, HID_DIM), jnp.float32),
            jax.ShapeDtypeStruct((N_NODES, 16), jnp.float32),
        ],
    )(p0, p1, h1)


def _dense2_body(p0_ref, p1_ref, g1_ref, dis_ref, b_ref, w_ref, g2_ref):
    dis = dis_ref[...]
    h = dis * (p0_ref[...] + p1_ref[...] + g1_ref[...]) + b_ref[...]
    h = jnp.maximum(h, 0.0)
    g2_ref[...] = jnp.dot(h, w_ref[...], preferred_element_type=jnp.float32) * dis


def _tc_dense2(p0, p1, g1, dis16, b1, W2):
    return pl.pallas_call(
        _dense2_body,
        grid=(GRID,),
        in_specs=[
            _rowspec(16), _rowspec(16), _rowspec(HID_DIM), _rowspec(16),
            pl.BlockSpec((1, 16), lambda i: (0, 0)),
            pl.BlockSpec((HID_DIM, OUT_DIM), lambda i: (0, 0)),
        ],
        out_specs=_rowspec(OUT_DIM),
        out_shape=jax.ShapeDtypeStruct((N_NODES, OUT_DIM), jnp.float32),
    )(p0, p1, g1, dis16, b1, W2)


def _dense3_body(q0_ref, q1_ref, g2_ref, dis_ref, b_ref, out_ref):
    out_ref[...] = (
        dis_ref[...] * (q0_ref[...] + q1_ref[...] + g2_ref[...]) + b_ref[...]
    )


def _tc_dense3(q0, q1, g2, dis16, b2):
    return pl.pallas_call(
        _dense3_body,
        grid=(GRID,),
        in_specs=[
            _rowspec(16), _rowspec(16), _rowspec(OUT_DIM), _rowspec(16),
            pl.BlockSpec((1, 16), lambda i: (0, 0)),
        ],
        out_specs=_rowspec(OUT_DIM),
        out_shape=jax.ShapeDtypeStruct((N_NODES, OUT_DIM), jnp.float32),
    )(q0, q1, g2, dis16, b2)


def kernel(x, edge_index, W1, b1, W2, b2):
    src = edge_index[0].astype(jnp.int32)
    dst = edge_index[1].astype(jnp.int32)
    pad = E_PAD - N_EDGES
    src_p = jnp.concatenate([src, jnp.zeros((pad,), jnp.int32)]).reshape(NW, JCH, CHUNK)
    dst_p = jnp.concatenate(
        [dst, jnp.full((pad,), N_NODES, jnp.int32)]
    ).reshape(NW, JCH, CHUNK)

    h1 = _tc_matmul1(x, W1)
    hp = _sc_hist(dst_p)
    g1, dis16 = _tc_scale1(hp[0, :N_NODES], hp[1, :N_NODES], h1)
    p = _sc_agg(g1, src_p, dst_p)
    g2 = _tc_dense2(
        p[0, :N_NODES], p[1, :N_NODES], g1, dis16, b1.reshape(1, 16), W2
    )
    q = _sc_agg(g2, src_p, dst_p)
    out = _tc_dense3(q[0, :N_NODES], q[1, :N_NODES], g2, dis16, b2.reshape(1, 16))
    return out
